# baseline (speedup 1.0000x reference)
import functools

import numpy as np
import jax
import jax.numpy as jnp
from jax import lax
from jax.experimental import pallas as pl
from jax.experimental.pallas import tpu as pltpu
from jax.experimental.pallas import tpu_sc as plsc

N = 10000
E = 320000
D = 128

NC = 2
NS = 16
NW = NC * NS
C = 128
STEPS = 80
QS = 16
NQ = STEPS // QS
EPT = C * STEPS
E_PAD = EPT * NW
N_PAD = 10240
RPT = N_PAD // NS

_PAD_IDX = np.asarray(N + np.arange(E_PAD - E) % (N_PAD - N), np.int32)

_mesh = plsc.VectorSubcoreMesh(core_axis_name="c", subcore_axis_name="s")


@functools.partial(
    pl.kernel,
    out_type=jax.ShapeDtypeStruct((NC, N_PAD), jnp.float32),
    mesh=_mesh,
    scratch_types=[
        pltpu.VMEM((STEPS, C), jnp.int32),
        pltpu.VMEM((C,), jnp.float32),
        pltpu.VMEM((C,), jnp.float32),
        pltpu.VMEM_SHARED((N_PAD,), jnp.float32),
        pltpu.SemaphoreType.DMA,
    ],
)
def _deg_kernel(col_hbm, out_hbm, col_v, ones_v, zeros_v, deg_sh, sem):
    cid = lax.axis_index("c")
    sid = lax.axis_index("s")
    wid = cid * NS + sid
    for i in range(C // 16):
        ones_v[pl.ds(i * 16, 16)] = jnp.ones((16,), jnp.float32)
        zeros_v[pl.ds(i * 16, 16)] = jnp.zeros((16,), jnp.float32)
    for j in range(RPT // C):
        pltpu.sync_copy(zeros_v, deg_sh.at[pl.ds(sid * RPT + j * C, C)])
    plsc.subcore_barrier()
    pltpu.sync_copy(col_hbm.at[pl.ds(wid * STEPS, STEPS)], col_v)

    def body(step, _):
        pltpu.async_copy(ones_v, deg_sh.at[col_v.at[step]], sem, add=True)
        return _

    lax.fori_loop(0, STEPS, body, None)

    def drain(step, _):
        pltpu.make_async_copy(ones_v, deg_sh.at[col_v.at[step]], sem).wait()
        return _

    lax.fori_loop(0, STEPS, drain, None)
    plsc.subcore_barrier()
    pltpu.sync_copy(deg_sh.at[pl.ds(sid * RPT, RPT)],
                    out_hbm.at[cid, pl.ds(sid * RPT, RPT)])


@functools.partial(
    pl.kernel,
    out_type=jax.ShapeDtypeStruct((NC, N_PAD, D), jnp.float32),
    mesh=_mesh,
    scratch_types=[
        pltpu.VMEM((QS, C), jnp.int32),
        pltpu.VMEM((QS, C), jnp.int32),
        pltpu.VMEM((C, D), jnp.float32),
        pltpu.VMEM((C, D), jnp.float32),
        pltpu.VMEM_SHARED((N_PAD, D), jnp.float32),
        pltpu.SemaphoreType.DMA,
        pltpu.SemaphoreType.DMA,
    ],
)
def _agg_kernel(s_hbm, row_hbm, col_hbm, out_hbm,
                row_v, col_v, g_a, g_b, agg_sh, sem_a, sem_b):
    cid = lax.axis_index("c")
    sid = lax.axis_index("s")
    wid = cid * NS + sid

    def zrow(i, _):
        for j in range(D // 16):
            g_a[i, pl.ds(j * 16, 16)] = jnp.zeros((16,), jnp.float32)
        return _

    lax.fori_loop(0, C, zrow, None)

    def zbody(j, _):
        pltpu.sync_copy(g_a, agg_sh.at[pl.ds(sid * RPT + j * C, C)])
        return _

    lax.fori_loop(0, RPT // C, zbody, None)
    plsc.subcore_barrier()

    def body(i, _):
        pltpu.make_async_copy(s_hbm.at[row_v.at[2 * i + 1]], g_b,
                              sem_b).start()
        pltpu.make_async_copy(s_hbm.at[row_v.at[2 * i]], g_a, sem_a).wait()
        pltpu.sync_copy(g_a, agg_sh.at[col_v.at[2 * i]], add=True)

        @pl.when(i < QS // 2 - 1)
        def _():
            pltpu.make_async_copy(s_hbm.at[row_v.at[2 * i + 2]], g_a,
                                  sem_a).start()

        pltpu.make_async_copy(s_hbm.at[row_v.at[2 * i + 1]], g_b,
                              sem_b).wait()
        pltpu.sync_copy(g_b, agg_sh.at[col_v.at[2 * i + 1]], add=True)
        return _

    for q in range(NQ):
        pltpu.sync_copy(row_hbm.at[pl.ds(wid * STEPS + q * QS, QS)], row_v)
        pltpu.sync_copy(col_hbm.at[pl.ds(wid * STEPS + q * QS, QS)], col_v)
        pltpu.make_async_copy(s_hbm.at[row_v.at[0]], g_a, sem_a).start()
        lax.fori_loop(0, QS // 2, body, None)

    plsc.subcore_barrier()
    pltpu.sync_copy(agg_sh.at[pl.ds(sid * RPT, RPT)],
                    out_hbm.at[cid, pl.ds(sid * RPT, RPT)])


def _mm_body(x_ref, w_ref, deg_ref, s_ref):
    deg = (deg_ref[0:1, :] + deg_ref[1:2, :]).reshape(N_PAD, 1)
    dis = jnp.where(deg > 0.0, lax.rsqrt(deg), 0.0)
    h = lax.dot_general(x_ref[...], w_ref[...],
                        (((1,), (1,)), ((), ())),
                        preferred_element_type=jnp.float32)
    s_ref[:N] = h * dis[:N]
    s_ref[N:] = jnp.zeros((N_PAD - N, D), jnp.float32)


EB = 2000
NEB = N // EB


def _final_body(x_ref, agg_ref, deg_ref, b_ref, g_ref, beta_ref, o_ref,
                pre_ref, acc_ref, dis_ref):
    i = pl.program_id(0)

    @pl.when(i == 0)
    def _():
        deg = (deg_ref[0:1, :] + deg_ref[1:2, :]).reshape(N_PAD, 1)
        dis_ref[...] = jnp.where(deg > 0.0, lax.rsqrt(deg), 0.0)

    @pl.when(i < NEB)
    def _():
        agg = agg_ref[0] + agg_ref[1]
        dis = dis_ref[pl.ds(i * EB, EB), :]
        pre = agg * dis + b_ref[...]
        pre_ref[pl.ds(i * EB, EB), :] = pre
        psum = jnp.sum(pre, axis=0, keepdims=True)
        psq = jnp.sum(pre * pre, axis=0, keepdims=True)

        @pl.when(i == 0)
        def _():
            acc_ref[0:1, :] = psum
            acc_ref[1:2, :] = psq

        @pl.when(i > 0)
        def _():
            acc_ref[0:1, :] += psum
            acc_ref[1:2, :] += psq

    @pl.when(i == NEB)
    def _():
        mean = acc_ref[0:1, :] * (1.0 / N)
        var = acc_ref[1:2, :] * (1.0 / N) - mean * mean
        scale = lax.rsqrt(var + 1e-5) * g_ref[...]
        shift = beta_ref[...] - mean * scale
        norm = pre_ref[...] * scale + shift
        o_ref[...] = x_ref[...] + jnp.maximum(norm, 0.0)


def kernel(x, edge_index, W, b, gamma, beta):
    pad = jnp.asarray(_PAD_IDX)
    col_r = jnp.concatenate([edge_index[1], pad]).reshape(NW * STEPS, C)
    row_r = jnp.concatenate([edge_index[0], pad]).reshape(NW * STEPS, C)

    deg_pair = _deg_kernel(col_r)

    s = pl.pallas_call(
        _mm_body,
        out_shape=jax.ShapeDtypeStruct((N_PAD, D), jnp.float32),
    )(x, W, deg_pair)

    agg_pair = _agg_kernel(s, row_r, col_r)

    out = pl.pallas_call(
        _final_body,
        grid=(NEB + 1,),
        in_specs=[
            pl.BlockSpec((N, D), lambda i: (0, 0)),
            pl.BlockSpec((NC, EB, D),
                         lambda i: (0, jax.lax.min(i, NEB - 1), 0)),
            pl.BlockSpec((NC, N_PAD), lambda i: (0, 0)),
            pl.BlockSpec((1, D), lambda i: (0, 0)),
            pl.BlockSpec((1, D), lambda i: (0, 0)),
            pl.BlockSpec((1, D), lambda i: (0, 0)),
        ],
        out_specs=pl.BlockSpec((N, D), lambda i: (0, 0)),
        scratch_shapes=[
            pltpu.VMEM((N, D), jnp.float32),
            pltpu.VMEM((2, D), jnp.float32),
            pltpu.VMEM((N_PAD, 1), jnp.float32),
        ],
        out_shape=jax.ShapeDtypeStruct((N, D), jnp.float32),
    )(x, agg_pair, deg_pair, b.reshape(1, D), gamma.reshape(1, D),
      beta.reshape(1, D))
    return out


# speedup vs baseline: 1.0022x; 1.0022x over previous
import functools

import numpy as np
import jax
import jax.numpy as jnp
from jax import lax
from jax.experimental import pallas as pl
from jax.experimental.pallas import tpu as pltpu
from jax.experimental.pallas import tpu_sc as plsc

N = 10000
E = 320000
D = 128

NC = 2
NS = 16
NW = NC * NS
C = 128
STEPS = 80
QS = 16
NQ = STEPS // QS
EPT = C * STEPS
E_PAD = EPT * NW
N_PAD = 10240
RPT = N_PAD // NS

_PAD_IDX = np.asarray(N + np.arange(E_PAD - E) % (N_PAD - N), np.int32)

_mesh = plsc.VectorSubcoreMesh(core_axis_name="c", subcore_axis_name="s")


@functools.partial(
    pl.kernel,
    out_type=jax.ShapeDtypeStruct((NC, N_PAD), jnp.float32),
    mesh=_mesh,
    scratch_types=[
        pltpu.VMEM((STEPS, C), jnp.int32),
        pltpu.VMEM((C,), jnp.float32),
        pltpu.VMEM((C,), jnp.float32),
        pltpu.VMEM_SHARED((N_PAD,), jnp.float32),
        pltpu.SemaphoreType.DMA,
    ],
)
def _deg_kernel(col_hbm, out_hbm, col_v, ones_v, zeros_v, deg_sh, sem):
    cid = lax.axis_index("c")
    sid = lax.axis_index("s")
    wid = cid * NS + sid
    for i in range(C // 16):
        ones_v[pl.ds(i * 16, 16)] = jnp.ones((16,), jnp.float32)
        zeros_v[pl.ds(i * 16, 16)] = jnp.zeros((16,), jnp.float32)
    for j in range(RPT // C):
        pltpu.sync_copy(zeros_v, deg_sh.at[pl.ds(sid * RPT + j * C, C)])
    plsc.subcore_barrier()
    pltpu.sync_copy(col_hbm.at[pl.ds(wid * STEPS, STEPS)], col_v)

    def body(step, _):
        pltpu.async_copy(ones_v, deg_sh.at[col_v.at[step]], sem, add=True)
        return _

    lax.fori_loop(0, STEPS, body, None)

    def drain(step, _):
        pltpu.make_async_copy(ones_v, deg_sh.at[col_v.at[step]], sem).wait()
        return _

    lax.fori_loop(0, STEPS, drain, None)
    plsc.subcore_barrier()
    pltpu.sync_copy(deg_sh.at[pl.ds(sid * RPT, RPT)],
                    out_hbm.at[cid, pl.ds(sid * RPT, RPT)])


@functools.partial(
    pl.kernel,
    out_type=jax.ShapeDtypeStruct((NC, N_PAD, D), jnp.float32),
    mesh=_mesh,
    scratch_types=[
        pltpu.VMEM((QS, C), jnp.int32),
        pltpu.VMEM((QS, C), jnp.int32),
        pltpu.VMEM((C, D), jnp.float32),
        pltpu.VMEM((C, D), jnp.float32),
        pltpu.VMEM_SHARED((N_PAD, D), jnp.float32),
        pltpu.SemaphoreType.DMA,
        pltpu.SemaphoreType.DMA,
    ],
)
def _agg_kernel(s_hbm, row_hbm, col_hbm, out_hbm,
                row_v, col_v, g_a, g_b, agg_sh, sem_a, sem_b):
    cid = lax.axis_index("c")
    sid = lax.axis_index("s")
    wid = cid * NS + sid

    def zrow(i, _):
        for j in range(D // 16):
            g_a[i, pl.ds(j * 16, 16)] = jnp.zeros((16,), jnp.float32)
        return _

    lax.fori_loop(0, C, zrow, None)

    def zbody(j, _):
        pltpu.sync_copy(g_a, agg_sh.at[pl.ds(sid * RPT + j * C, C)])
        return _

    lax.fori_loop(0, RPT // C, zbody, None)
    plsc.subcore_barrier()

    def body(i, _):
        pltpu.make_async_copy(s_hbm.at[row_v.at[2 * i + 1]], g_b,
                              sem_b).start()
        pltpu.make_async_copy(s_hbm.at[row_v.at[2 * i]], g_a, sem_a).wait()
        pltpu.sync_copy(g_a, agg_sh.at[col_v.at[2 * i]], add=True)

        @pl.when(i < QS // 2 - 1)
        def _():
            pltpu.make_async_copy(s_hbm.at[row_v.at[2 * i + 2]], g_a,
                                  sem_a).start()

        pltpu.make_async_copy(s_hbm.at[row_v.at[2 * i + 1]], g_b,
                              sem_b).wait()
        pltpu.sync_copy(g_b, agg_sh.at[col_v.at[2 * i + 1]], add=True)
        return _

    for q in range(NQ):
        pltpu.sync_copy(row_hbm.at[pl.ds(wid * STEPS + q * QS, QS)], row_v)
        pltpu.sync_copy(col_hbm.at[pl.ds(wid * STEPS + q * QS, QS)], col_v)
        pltpu.make_async_copy(s_hbm.at[row_v.at[0]], g_a, sem_a).start()
        lax.fori_loop(0, QS // 2, body, None)

    plsc.subcore_barrier()
    pltpu.sync_copy(agg_sh.at[pl.ds(sid * RPT, RPT)],
                    out_hbm.at[cid, pl.ds(sid * RPT, RPT)])


def _mm_body(x_ref, w_ref, deg_ref, s_ref):
    deg = (deg_ref[0:1, :] + deg_ref[1:2, :]).reshape(N_PAD, 1)
    dis = jnp.where(deg > 0.0, lax.rsqrt(deg), 0.0)
    h = lax.dot_general(x_ref[...], w_ref[...],
                        (((1,), (1,)), ((), ())),
                        preferred_element_type=jnp.float32)
    s_ref[:N] = h * dis[:N]
    s_ref[N:] = jnp.zeros((N_PAD - N, D), jnp.float32)


def _final_body(x_ref, agg_ref, deg_ref, b_ref, g_ref, beta_ref, o_ref):
    agg = agg_ref[0, :N, :] + agg_ref[1, :N, :]
    deg = (deg_ref[0:1, :] + deg_ref[1:2, :]).reshape(N_PAD, 1)[:N]
    dis = jnp.where(deg > 0.0, lax.rsqrt(deg), 0.0)
    pre = agg * dis + b_ref[...]
    mean = jnp.mean(pre, axis=0, keepdims=True)
    cent = pre - mean
    var = jnp.mean(cent * cent, axis=0, keepdims=True)
    norm = cent * lax.rsqrt(var + 1e-5) * g_ref[...] + beta_ref[...]
    o_ref[...] = x_ref[...] + jnp.maximum(norm, 0.0)


def kernel(x, edge_index, W, b, gamma, beta):
    pad = jnp.asarray(_PAD_IDX)
    col_r = jnp.concatenate([edge_index[1], pad]).reshape(NW * STEPS, C)
    row_r = jnp.concatenate([edge_index[0], pad]).reshape(NW * STEPS, C)

    deg_pair = _deg_kernel(col_r)

    s = pl.pallas_call(
        _mm_body,
        out_shape=jax.ShapeDtypeStruct((N_PAD, D), jnp.float32),
    )(x, W, deg_pair)

    agg_pair = _agg_kernel(s, row_r, col_r)

    out = pl.pallas_call(
        _final_body,
        out_shape=jax.ShapeDtypeStruct((N, D), jnp.float32),
    )(x, agg_pair, deg_pair, b.reshape(1, D), gamma.reshape(1, D),
      beta.reshape(1, D))
    return out


# speedup vs baseline: 1.0376x; 1.0353x over previous
import functools

import numpy as np
import jax
import jax.numpy as jnp
from jax import lax
from jax.experimental import pallas as pl
from jax.experimental.pallas import tpu as pltpu
from jax.experimental.pallas import tpu_sc as plsc

N = 10000
E = 320000
D = 128

NC = 2
NS = 16
NW = NC * NS
C = 128
STEPS = 80
QS = 16
NQ = STEPS // QS
EPT = C * STEPS
E_PAD = EPT * NW
N_PAD = 10240
RPT = N_PAD // NS

_PAD_IDX = np.broadcast_to(
    np.asarray(N + np.arange(E_PAD - E) % (N_PAD - N), np.int32),
    (2, E_PAD - E))

_mesh = plsc.VectorSubcoreMesh(core_axis_name="c", subcore_axis_name="s")


@functools.partial(
    pl.kernel,
    out_type=jax.ShapeDtypeStruct((NC, N_PAD), jnp.float32),
    mesh=_mesh,
    scratch_types=[
        pltpu.VMEM((STEPS, C), jnp.int32),
        pltpu.VMEM((C,), jnp.float32),
        pltpu.VMEM((C,), jnp.float32),
        pltpu.VMEM_SHARED((N_PAD,), jnp.float32),
        pltpu.SemaphoreType.DMA,
    ],
)
def _deg_kernel(ei_hbm, out_hbm, col_v, ones_v, zeros_v, deg_sh, sem):
    cid = lax.axis_index("c")
    sid = lax.axis_index("s")
    wid = cid * NS + sid
    for i in range(C // 16):
        ones_v[pl.ds(i * 16, 16)] = jnp.ones((16,), jnp.float32)
        zeros_v[pl.ds(i * 16, 16)] = jnp.zeros((16,), jnp.float32)
    for j in range(RPT // C):
        pltpu.sync_copy(zeros_v, deg_sh.at[pl.ds(sid * RPT + j * C, C)])
    plsc.subcore_barrier()
    pltpu.sync_copy(ei_hbm.at[1, pl.ds(wid * STEPS, STEPS)], col_v)

    def body(step, _):
        pltpu.async_copy(ones_v, deg_sh.at[col_v.at[step]], sem, add=True)
        return _

    lax.fori_loop(0, STEPS, body, None)

    def drain(step, _):
        pltpu.make_async_copy(ones_v, deg_sh.at[col_v.at[step]], sem).wait()
        return _

    lax.fori_loop(0, STEPS, drain, None)
    plsc.subcore_barrier()
    pltpu.sync_copy(deg_sh.at[pl.ds(sid * RPT, RPT)],
                    out_hbm.at[cid, pl.ds(sid * RPT, RPT)])


@functools.partial(
    pl.kernel,
    out_type=jax.ShapeDtypeStruct((NC, N_PAD, D), jnp.float32),
    mesh=_mesh,
    scratch_types=[
        pltpu.VMEM((QS, C), jnp.int32),
        pltpu.VMEM((QS, C), jnp.int32),
        pltpu.VMEM((C, D), jnp.float32),
        pltpu.VMEM((C, D), jnp.float32),
        pltpu.VMEM_SHARED((N_PAD, D), jnp.float32),
        pltpu.SemaphoreType.DMA,
        pltpu.SemaphoreType.DMA,
    ],
)
def _agg_kernel(s_hbm, ei_hbm, out_hbm,
                row_v, col_v, g_a, g_b, agg_sh, sem_a, sem_b):
    cid = lax.axis_index("c")
    sid = lax.axis_index("s")
    wid = cid * NS + sid

    def zrow(i, _):
        for j in range(D // 16):
            g_a[i, pl.ds(j * 16, 16)] = jnp.zeros((16,), jnp.float32)
        return _

    lax.fori_loop(0, C, zrow, None)

    def zbody(j, _):
        pltpu.sync_copy(g_a, agg_sh.at[pl.ds(sid * RPT + j * C, C)])
        return _

    lax.fori_loop(0, RPT // C, zbody, None)
    plsc.subcore_barrier()

    def body(i, _):
        pltpu.make_async_copy(s_hbm.at[row_v.at[2 * i + 1]], g_b,
                              sem_b).start()
        pltpu.make_async_copy(s_hbm.at[row_v.at[2 * i]], g_a, sem_a).wait()
        pltpu.sync_copy(g_a, agg_sh.at[col_v.at[2 * i]], add=True)

        @pl.when(i < QS // 2 - 1)
        def _():
            pltpu.make_async_copy(s_hbm.at[row_v.at[2 * i + 2]], g_a,
                                  sem_a).start()

        pltpu.make_async_copy(s_hbm.at[row_v.at[2 * i + 1]], g_b,
                              sem_b).wait()
        pltpu.sync_copy(g_b, agg_sh.at[col_v.at[2 * i + 1]], add=True)
        return _

    for q in range(NQ):
        pltpu.sync_copy(ei_hbm.at[0, pl.ds(wid * STEPS + q * QS, QS)], row_v)
        pltpu.sync_copy(ei_hbm.at[1, pl.ds(wid * STEPS + q * QS, QS)], col_v)
        pltpu.make_async_copy(s_hbm.at[row_v.at[0]], g_a, sem_a).start()
        lax.fori_loop(0, QS // 2, body, None)

    plsc.subcore_barrier()
    pltpu.sync_copy(agg_sh.at[pl.ds(sid * RPT, RPT)],
                    out_hbm.at[cid, pl.ds(sid * RPT, RPT)])


def _mm_body(x_ref, w_ref, deg_ref, s_ref):
    deg = (deg_ref[0:1, :] + deg_ref[1:2, :]).reshape(N_PAD, 1)
    dis = jnp.where(deg > 0.0, lax.rsqrt(deg), 0.0)
    h = lax.dot_general(x_ref[...], w_ref[...],
                        (((1,), (1,)), ((), ())),
                        preferred_element_type=jnp.float32)
    s_ref[:N] = h * dis[:N]
    s_ref[N:] = jnp.zeros((N_PAD - N, D), jnp.float32)


def _final_body(x_ref, agg_ref, deg_ref, b_ref, g_ref, beta_ref, o_ref):
    agg = agg_ref[0, :N, :] + agg_ref[1, :N, :]
    deg = (deg_ref[0:1, :] + deg_ref[1:2, :]).reshape(N_PAD, 1)[:N]
    dis = jnp.where(deg > 0.0, lax.rsqrt(deg), 0.0)
    pre = agg * dis + b_ref[...]
    mean = jnp.mean(pre, axis=0, keepdims=True)
    cent = pre - mean
    var = jnp.mean(cent * cent, axis=0, keepdims=True)
    norm = cent * lax.rsqrt(var + 1e-5) * g_ref[...] + beta_ref[...]
    o_ref[...] = x_ref[...] + jnp.maximum(norm, 0.0)


def kernel(x, edge_index, W, b, gamma, beta):
    ei = jnp.concatenate([edge_index, jnp.asarray(_PAD_IDX)], axis=1)
    ei = ei.reshape(2, NW * STEPS, C)

    deg_pair = _deg_kernel(ei)

    s = pl.pallas_call(
        _mm_body,
        out_shape=jax.ShapeDtypeStruct((N_PAD, D), jnp.float32),
    )(x, W, deg_pair)

    agg_pair = _agg_kernel(s, ei)

    out = pl.pallas_call(
        _final_body,
        out_shape=jax.ShapeDtypeStruct((N, D), jnp.float32),
    )(x, agg_pair, deg_pair, b.reshape(1, D), gamma.reshape(1, D),
      beta.reshape(1, D))
    return out
